# baseline (speedup 1.0000x reference)
import functools
import math

import jax
import jax.numpy as jnp
from jax import lax
from jax.experimental import pallas as pl
from jax.experimental.pallas import tpu as pltpu
from jax.experimental.pallas import tpu_sc as plsc

BATCH = 4096
HIST = 200
EMB_DIM = 32
VOC = 1000000
TOTAL = BATCH * HIST
SCALE = math.sqrt(EMB_DIM)

_info = plsc.get_sparse_core_info()
NC = _info.num_cores
NS = _info.num_subcores
LANES = _info.num_lanes
NW = NC * NS

_mesh = plsc.VectorSubcoreMesh(core_axis_name="c", subcore_axis_name="s")


RBLK = 512
N_FULL = (VOC // RBLK)
UNITS_EACH = N_FULL // NW
UNITS_REM = N_FULL % NW
TAIL_R0 = N_FULL * RBLK
TAIL_N = VOC - TAIL_R0


@functools.partial(
    pl.kernel,
    mesh=_mesh,
    compiler_params=pltpu.CompilerParams(
        use_tc_tiling_on_sc=True, needs_layout_passes=False
    ),
    out_type=jax.ShapeDtypeStruct((VOC * EMB_DIM,), jnp.float32),
    scratch_types=[
        [pltpu.VMEM((EMB_DIM, RBLK + 1), jnp.float32) for _ in range(2)],
        [pltpu.VMEM((RBLK * EMB_DIM,), jnp.float32) for _ in range(2)],
        pltpu.VMEM((TAIL_N * EMB_DIM,), jnp.float32),
        [pltpu.SemaphoreType.DMA for _ in range(2)],
        [pltpu.SemaphoreType.DMA for _ in range(2)],
        pltpu.SemaphoreType.DMA,
    ],
)
def _untile(wt_hbm, wtail_hbm, wrow_hbm, tv, ov, tov, gsems, wsems, tsem):
    wid = lax.axis_index("s") * NC + lax.axis_index("c")
    iota = lax.iota(jnp.int32, LANES)
    iota_hi = iota + LANES

    def col_of(k):
        return k * NW + wid

    def start_read(k, p):
        tc = col_of(k)
        for tr in range(EMB_DIM // 8):
            pltpu.async_copy(
                wt_hbm.at[pl.ds(tr * 8, 8), pl.ds(tc * RBLK, RBLK)],
                tv[p].at[pl.ds(tr * 8, 8), pl.ds(0, RBLK)],
                gsems[p],
            )

    def wait_read(p):
        pltpu.make_async_copy(
            wt_hbm.at[:, pl.ds(0, RBLK)],
            tv[p].at[:, pl.ds(0, RBLK)],
            gsems[p],
        ).wait()

    def untile_block(p):
        @plsc.parallel_loop(0, RBLK, unroll=8)
        def _(rs):
            cs = jnp.full((LANES,), rs, jnp.int32)
            lo = plsc.load_gather(tv[p], [iota, cs])
            hi = plsc.load_gather(tv[p], [iota_hi, cs])
            ov[p][pl.ds(rs * EMB_DIM, LANES)] = lo
            ov[p][pl.ds(rs * EMB_DIM + LANES, LANES)] = hi

    def start_write(k, p):
        tc = col_of(k)
        pltpu.async_copy(
            ov[p], wrow_hbm.at[pl.ds(tc * (RBLK * EMB_DIM), RBLK * EMB_DIM)],
            wsems[p],
        )

    def wait_write(p):
        pltpu.make_async_copy(
            ov[p], wrow_hbm.at[pl.ds(0, RBLK * EMB_DIM)], wsems[p]
        ).wait()

    n_units = UNITS_EACH + 1
    start_read(0, 0)

    def body(k, carry):
        for p in range(2):
            kk = k * 2 + p

            @pl.when(kk < UNITS_EACH)
            def _():
                @pl.when(kk + 1 < n_units)
                def _():
                    do_next = jnp.logical_or(
                        kk + 1 < UNITS_EACH, wid < UNITS_REM
                    )

                    @pl.when(do_next)
                    def _():
                        start_read(kk + 1, 1 - p)

                wait_read(p)

                @pl.when(kk >= 2)
                def _():
                    wait_write(p)

                untile_block(p)
                start_write(kk, p)
        return carry

    lax.fori_loop(0, (UNITS_EACH + 1) // 2, body, 0)

    p_x = UNITS_EACH % 2

    @pl.when(wid < UNITS_REM)
    def _():
        wait_read(p_x)
        wait_write(p_x)
        untile_block(p_x)
        start_write(UNITS_EACH, p_x)

    @pl.when(wid == NW - 1)
    def _():
        pltpu.sync_copy(wtail_hbm, tov)
        pltpu.async_copy(
            tov, wrow_hbm.at[pl.ds(TAIL_R0 * EMB_DIM, TAIL_N * EMB_DIM)],
            tsem,
        )
        pltpu.make_async_copy(
            tov, wrow_hbm.at[pl.ds(0, TAIL_N * EMB_DIM)], tsem
        ).wait()

    for p in range(2):
        wait_write(p)


PER_W = TOTAL // NW
HBLK = 128
N_TR = EMB_DIM // 8
N_TC = BATCH // HBLK
CHB = 512
TCPU = CHB // HBLK
UNITS_B = PER_W // CHB
UPH = BATCH // CHB


@functools.partial(
    pl.kernel,
    mesh=_mesh,
    compiler_params=pltpu.CompilerParams(
        use_tc_tiling_on_sc=False, needs_layout_passes=False
    ),
    out_type=jax.ShapeDtypeStruct(
        (HIST, N_TR, N_TC, 8, HBLK), jnp.float32
    ),
    scratch_types=[
        pltpu.VMEM((PER_W,), jnp.int32),
        [pltpu.VMEM((CHB, EMB_DIM), jnp.float32) for _ in range(2)],
        [pltpu.VMEM((N_TR * TCPU * 8, HBLK + 1), jnp.float32) for _ in range(2)],
        [pltpu.SemaphoreType.DMA for _ in range(2)],
        [pltpu.SemaphoreType.DMA for _ in range(2)],
    ],
)
def _embed(w_hbm, x_hbm, out_hbm, idx_v, rbuf, tbuf, gsems, ssems):
    wid = lax.axis_index("s") * NC + lax.axis_index("c")
    base = wid * PER_W
    iota = lax.iota(jnp.int32, LANES)
    row_lo = lax.shift_right_logical(iota, 3) * (TCPU * 8) + (iota & 7)
    row_hi = row_lo + 2 * (TCPU * 8)

    pltpu.sync_copy(x_hbm.at[pl.ds(base, PER_W)], idx_v)

    def start_gather(k, p):
        pltpu.async_copy(
            w_hbm.at[idx_v.at[pl.ds(k * CHB, CHB)]], rbuf[p], gsems[p]
        )

    def wait_gather(p):
        pltpu.make_async_copy(
            w_hbm.at[idx_v.at[pl.ds(0, CHB)]], rbuf[p], gsems[p]
        ).wait()

    def transpose_scale(p):
        @plsc.parallel_loop(0, CHB, unroll=8)
        def _(j):
            tcl8 = lax.shift_right_logical(j, 7) * 8
            cs = jnp.full((LANES,), j & (HBLK - 1), jnp.int32)
            lo = rbuf[p][j, pl.ds(0, LANES)] * SCALE
            hi = rbuf[p][j, pl.ds(LANES, LANES)] * SCALE
            plsc.store_scatter(tbuf[p], [row_lo + tcl8, cs], lo)
            plsc.store_scatter(tbuf[p], [row_hi + tcl8, cs], hi)

    def start_store(k, p):
        u = wid * UNITS_B + k
        h = u // UPH
        tc0 = (u % UPH) * TCPU
        for tr in range(N_TR):
            for tcl in range(TCPU):
                pltpu.async_copy(
                    tbuf[p].at[
                        pl.ds(tr * (TCPU * 8) + tcl * 8, 8), pl.ds(0, HBLK)
                    ],
                    out_hbm.at[h, tr, tc0 + tcl, :, :],
                    ssems[p],
                )

    def wait_store(p):
        for _ in range(N_TR * TCPU):
            pltpu.make_async_copy(
                tbuf[p].at[pl.ds(0, 8), pl.ds(0, HBLK)],
                out_hbm.at[0, 0, 0, :, :],
                ssems[p],
            ).wait()

    start_gather(0, 0)

    def loop_body(t, carry):
        for p in range(2):
            k = t * 2 + p

            @pl.when(k + 1 < UNITS_B)
            def _():
                start_gather(k + 1, 1 - p)

            wait_gather(p)

            @pl.when(k >= 2)
            def _():
                wait_store(p)

            transpose_scale(p)
            start_store(k, p)
        return carry

    lax.fori_loop(0, UNITS_B // 2, loop_body, 0)

    for p in range(2):
        wait_store(p)


def kernel(x, W):
    wrow = _untile(jnp.transpose(W), W[TAIL_R0:].reshape(TAIL_N * EMB_DIM))
    out5 = _embed(wrow.reshape(VOC, EMB_DIM), jnp.transpose(x).reshape(TOTAL))
    return jnp.transpose(out5, (2, 4, 0, 1, 3)).reshape(BATCH, HIST, EMB_DIM)
